# baseline (speedup 1.0000x reference)
import functools

import jax
import jax.numpy as jnp
from jax import lax
from jax.experimental import pallas as pl
from jax.experimental.pallas import tpu as pltpu
from jax.experimental.pallas import tpu_sc as plsc

N = 10000
E = 320000
D = 128
H = 256

NC = 2
NS = 16
NW = NC * NS
EPW = E // NW
K = 80
CH = EPW // K

RSTRIDE = 624
RSPAN = 640
ZR = 128

_mesh = plsc.VectorSubcoreMesh(core_axis_name="c", subcore_axis_name="s",
                               num_cores=NC, num_subcores=NS)


@functools.partial(
    pl.kernel,
    out_type=jax.ShapeDtypeStruct((NC * N, D), jnp.float32),
    mesh=_mesh,
    scratch_types=(
        [pltpu.VMEM((2, K), jnp.int32) for _ in range(8)]
        + [pltpu.VMEM((K, D), jnp.float32) for _ in range(4)]
        + [pltpu.VMEM_SHARED((N, D), jnp.float32)]
        + [pltpu.SemaphoreType.DMA for _ in range(17)]
    ),
)
def _sc_agg(src_hbm, dst_hbm, x_hbm, zeros_hbm, out_hbm,
            ia0, ia1, ia2, ia3, ib0, ib1, ib2, ib3, r0, r1, r2, r3, agg_sh,
            zsem, la0, la1, la2, la3, lb0, lb1, lb2, lb3,
            g0, g1, g2, g3, s0, s1, s2, s3):
    c = lax.axis_index("c")
    s = lax.axis_index("s")
    wid = s * NC + c
    idx_a = [ia0, ia1, ia2, ia3]
    idx_b = [ib0, ib1, ib2, ib3]
    rows = [r0, r1, r2, r3]
    ils_a = [la0, la1, la2, la3]
    ils_b = [lb0, lb1, lb2, lb3]
    gs = [g0, g1, g2, g3]
    ss = [s0, s1, s2, s3]
    Q = 4

    def _load_idx(idx, ils, j, chunk):
        base = wid * EPW + chunk * K
        pltpu.async_copy(src_hbm.at[pl.ds(base, K)], idx[j].at[0], ils[j])
        pltpu.async_copy(dst_hbm.at[pl.ds(base, K)], idx[j].at[1], ils[j])

    def _prefetch(idx, ils, base):
        for j in range(Q):
            _load_idx(idx, ils, j, jnp.minimum(base + j, CH - 1))

    def _wait_idx(idx, ils, j):
        pltpu.make_async_copy(src_hbm.at[pl.ds(0, K)], idx[j].at[0],
                              ils[j]).wait()
        pltpu.make_async_copy(src_hbm.at[pl.ds(0, K)], idx[j].at[1],
                              ils[j]).wait()

    def _wait_gather(idx, j):
        pltpu.make_async_copy(x_hbm.at[idx[j].at[0]], rows[j], gs[j]).wait()

    def _half(idx_cur, ils_cur, idx_nxt, ils_nxt, base):
        for j in range(Q):
            _wait_gather(idx_cur, j)
            pltpu.async_copy(rows[j], agg_sh.at[idx_cur[j].at[1]], ss[j],
                             add=True)
        for j in range(Q):
            pltpu.make_async_copy(rows[j], agg_sh.at[idx_cur[j].at[1]],
                                  ss[j]).wait()
            _load_idx(idx_cur, ils_cur, j,
                      jnp.minimum(base + 2 * Q + j, CH - 1))
            _wait_idx(idx_nxt, ils_nxt, j)
            pltpu.async_copy(x_hbm.at[idx_nxt[j].at[0]], rows[j], gs[j])

    _prefetch(idx_a, ils_a, 0)
    _prefetch(idx_b, ils_b, Q)
    dz = [pltpu.async_copy(zeros_hbm,
                           agg_sh.at[pl.ds(s * RSTRIDE + j * ZR, ZR)], zsem)
          for j in range(RSPAN // ZR)]
    for j in range(Q):
        _wait_idx(idx_a, ils_a, j)
        pltpu.async_copy(x_hbm.at[idx_a[j].at[0]], rows[j], gs[j])
    for d in dz:
        d.wait()
    plsc.subcore_barrier()

    def _dbl_body(i, carry):
        base = 2 * Q * i
        _half(idx_a, ils_a, idx_b, ils_b, base)
        _half(idx_b, ils_b, idx_a, ils_a, base + Q)
        return carry

    lax.fori_loop(0, (CH - 1) // (2 * Q), _dbl_body, 0)
    for j in range(Q):
        _wait_gather(idx_a, j)
        pltpu.async_copy(rows[j], agg_sh.at[idx_a[j].at[1]], ss[j], add=True)
    for j in range(Q):
        pltpu.make_async_copy(rows[j], agg_sh.at[idx_a[j].at[1]],
                              ss[j]).wait()
        _wait_idx(idx_b, ils_b, j)
    pltpu.async_copy(x_hbm.at[idx_b[0].at[0]], rows[0], gs[0])
    _wait_gather(idx_b, 0)
    pltpu.sync_copy(rows[0], agg_sh.at[idx_b[0].at[1]], add=True)
    plsc.subcore_barrier()

    pltpu.sync_copy(agg_sh.at[pl.ds(s * RSTRIDE, RSPAN)],
                    out_hbm.at[pl.ds(c * N + s * RSTRIDE, RSPAN)])


def _untile_body(e_ref, s_ref, d_ref):
    s_ref[...] = jnp.reshape(e_ref[0, :], (E // 128, 128))
    d_ref[...] = jnp.reshape(e_ref[1, :], (E // 128, 128))


_untile = pl.pallas_call(
    _untile_body,
    out_shape=[jax.ShapeDtypeStruct((E // 128, 128), jnp.int32),
               jax.ShapeDtypeStruct((E // 128, 128), jnp.int32)],
)


BLK = 10000


def _dot(a, b):
    return jnp.dot(a, b, preferred_element_type=jnp.float32)


def _mlp_body(eps_ref, x_ref, a0_ref, a1_ref, w1_ref, b1_ref, w2_ref, b2_ref,
              w3_ref, b3_ref, o_ref):
    h = x_ref[...] * (1.0 + eps_ref[0]) + a0_ref[...] + a1_ref[...]
    h = jnp.maximum(_dot(h, w1_ref[...]) + b1_ref[...], 0.0)
    h = jnp.maximum(_dot(h, w2_ref[...]) + b2_ref[...], 0.0)
    o_ref[...] = _dot(h, w3_ref[...]) + b3_ref[...]


_mlp = pl.pallas_call(
    _mlp_body,
    grid=(N // BLK,),
    in_specs=[
        pl.BlockSpec(memory_space=pltpu.SMEM),
        pl.BlockSpec((BLK, D), lambda i: (i, 0)),
        pl.BlockSpec((BLK, D), lambda i: (i, 0)),
        pl.BlockSpec((BLK, D), lambda i: (N // BLK + i, 0)),
        pl.BlockSpec((D, H), lambda i: (0, 0)),
        pl.BlockSpec((1, H), lambda i: (0, 0)),
        pl.BlockSpec((H, H), lambda i: (0, 0)),
        pl.BlockSpec((1, H), lambda i: (0, 0)),
        pl.BlockSpec((H, D), lambda i: (0, 0)),
        pl.BlockSpec((1, D), lambda i: (0, 0)),
    ],
    out_specs=pl.BlockSpec((BLK, D), lambda i: (i, 0)),
    out_shape=jax.ShapeDtypeStruct((N, D), jnp.float32),
)


def kernel(edge_index, embed, eps0, W1_0, b1_0, W2_0, b2_0, W3_0, b3_0,
           eps1, W1_1, b1_1, W2_1, b2_1, W3_1, b3_1,
           eps2, W1_2, b1_2, W2_2, b2_2, W3_2, b3_2):
    zeros = jnp.zeros((ZR, D), jnp.float32)
    s2, d2 = _untile(edge_index)
    src = jnp.reshape(s2, (E,))
    dst = jnp.reshape(d2, (E,))
    x = embed
    params = [(eps0, W1_0, b1_0, W2_0, b2_0, W3_0, b3_0),
              (eps1, W1_1, b1_1, W2_1, b2_1, W3_1, b3_1),
              (eps2, W1_2, b1_2, W2_2, b2_2, W3_2, b3_2)]
    for eps, W1, b1, W2, b2, W3, b3 in params:
        parts = _sc_agg(src, dst, x, zeros)
        x = _mlp(jnp.reshape(eps, (1,)), x, parts, parts,
                 W1, jnp.reshape(b1, (1, H)),
                 W2, jnp.reshape(b2, (1, H)),
                 W3, jnp.reshape(b3, (1, D)))
    return x


# speedup vs baseline: 1.0158x; 1.0158x over previous
import functools

import jax
import jax.numpy as jnp
from jax import lax
from jax.experimental import pallas as pl
from jax.experimental.pallas import tpu as pltpu
from jax.experimental.pallas import tpu_sc as plsc

N = 10000
E = 320000
D = 128
H = 256

NC = 2
NS = 16
NW = NC * NS
EPW = E // NW
K = 80
CH = EPW // K

RSTRIDE = 624
RSPAN = 640
ZR = 128

_mesh = plsc.VectorSubcoreMesh(core_axis_name="c", subcore_axis_name="s",
                               num_cores=NC, num_subcores=NS)


@functools.partial(
    pl.kernel,
    out_type=jax.ShapeDtypeStruct((NC * N, D), jnp.float32),
    mesh=_mesh,
    scratch_types=(
        [pltpu.VMEM((2, K), jnp.int32) for _ in range(8)]
        + [pltpu.VMEM((K, D), jnp.float32) for _ in range(4)]
        + [pltpu.VMEM_SHARED((N, D), jnp.float32)]
        + [pltpu.SemaphoreType.DMA for _ in range(17)]
    ),
)
def _sc_agg(src_hbm, dst_hbm, x_hbm, zeros_hbm, out_hbm,
            ia0, ia1, ia2, ia3, ib0, ib1, ib2, ib3, r0, r1, r2, r3, agg_sh,
            zsem, la0, la1, la2, la3, lb0, lb1, lb2, lb3,
            g0, g1, g2, g3, s0, s1, s2, s3):
    c = lax.axis_index("c")
    s = lax.axis_index("s")
    wid = s * NC + c
    idx_a = [ia0, ia1, ia2, ia3]
    idx_b = [ib0, ib1, ib2, ib3]
    rows = [r0, r1, r2, r3]
    ils_a = [la0, la1, la2, la3]
    ils_b = [lb0, lb1, lb2, lb3]
    gs = [g0, g1, g2, g3]
    ss = [s0, s1, s2, s3]
    Q = 4

    def _load_idx(idx, ils, j, chunk):
        base = wid * EPW + chunk * K
        pltpu.async_copy(src_hbm.at[pl.ds(base, K)], idx[j].at[0], ils[j])
        pltpu.async_copy(dst_hbm.at[pl.ds(base, K)], idx[j].at[1], ils[j])

    def _prefetch(idx, ils, base):
        for j in range(Q):
            _load_idx(idx, ils, j, jnp.minimum(base + j, CH - 1))

    def _wait_idx(idx, ils, j):
        pltpu.make_async_copy(src_hbm.at[pl.ds(0, K)], idx[j].at[0],
                              ils[j]).wait()
        pltpu.make_async_copy(src_hbm.at[pl.ds(0, K)], idx[j].at[1],
                              ils[j]).wait()

    def _wait_gather(idx, j):
        pltpu.make_async_copy(x_hbm.at[idx[j].at[0]], rows[j], gs[j]).wait()

    def _half(idx_cur, ils_cur, idx_nxt, ils_nxt, base):
        for j in range(Q):
            _wait_gather(idx_cur, j)
            pltpu.async_copy(rows[j], agg_sh.at[idx_cur[j].at[1]], ss[j],
                             add=True)
        for j in range(Q):
            pltpu.make_async_copy(rows[j], agg_sh.at[idx_cur[j].at[1]],
                                  ss[j]).wait()
            _load_idx(idx_cur, ils_cur, j,
                      jnp.minimum(base + 2 * Q + j, CH - 1))
            _wait_idx(idx_nxt, ils_nxt, j)
            pltpu.async_copy(x_hbm.at[idx_nxt[j].at[0]], rows[j], gs[j])

    _prefetch(idx_a, ils_a, 0)
    _prefetch(idx_b, ils_b, Q)
    dz = [pltpu.async_copy(zeros_hbm,
                           agg_sh.at[pl.ds(s * RSTRIDE + j * ZR, ZR)], zsem)
          for j in range(RSPAN // ZR)]
    for j in range(Q):
        _wait_idx(idx_a, ils_a, j)
        pltpu.async_copy(x_hbm.at[idx_a[j].at[0]], rows[j], gs[j])
    for d in dz:
        d.wait()
    plsc.subcore_barrier()

    def _dbl_body(i, carry):
        base = 2 * Q * i
        _half(idx_a, ils_a, idx_b, ils_b, base)
        _half(idx_b, ils_b, idx_a, ils_a, base + Q)
        return carry

    lax.fori_loop(0, (CH - 1) // (2 * Q), _dbl_body, 0)
    for j in range(Q):
        _wait_gather(idx_a, j)
        pltpu.async_copy(rows[j], agg_sh.at[idx_a[j].at[1]], ss[j], add=True)
    for j in range(Q):
        pltpu.make_async_copy(rows[j], agg_sh.at[idx_a[j].at[1]],
                              ss[j]).wait()
        _wait_idx(idx_b, ils_b, j)
    pltpu.async_copy(x_hbm.at[idx_b[0].at[0]], rows[0], gs[0])
    _wait_gather(idx_b, 0)
    pltpu.sync_copy(rows[0], agg_sh.at[idx_b[0].at[1]], add=True)
    plsc.subcore_barrier()

    pltpu.sync_copy(agg_sh.at[pl.ds(s * RSTRIDE, RSPAN)],
                    out_hbm.at[pl.ds(c * N + s * RSTRIDE, RSPAN)])


def _untile_body(e_ref, s_ref, d_ref):
    s_ref[...] = jnp.reshape(e_ref[0, :], (E // 128, 128))
    d_ref[...] = jnp.reshape(e_ref[1, :], (E // 128, 128))


_untile = pl.pallas_call(
    _untile_body,
    out_shape=[jax.ShapeDtypeStruct((E // 128, 128), jnp.int32),
               jax.ShapeDtypeStruct((E // 128, 128), jnp.int32)],
)


BLK = 5000


def _dot(a, b):
    return jnp.dot(a, b, preferred_element_type=jnp.float32)


def _mlp_body(eps_ref, x_ref, a0_ref, a1_ref, w1_ref, b1_ref, w2_ref, b2_ref,
              w3_ref, b3_ref, o_ref):
    h = x_ref[...] * (1.0 + eps_ref[0]) + a0_ref[...] + a1_ref[...]
    h = jnp.maximum(_dot(h, w1_ref[...]) + b1_ref[...], 0.0)
    h = jnp.maximum(_dot(h, w2_ref[...]) + b2_ref[...], 0.0)
    o_ref[...] = _dot(h, w3_ref[...]) + b3_ref[...]


_mlp = pl.pallas_call(
    _mlp_body,
    grid=(N // BLK,),
    in_specs=[
        pl.BlockSpec(memory_space=pltpu.SMEM),
        pl.BlockSpec((BLK, D), lambda i: (i, 0)),
        pl.BlockSpec((BLK, D), lambda i: (i, 0)),
        pl.BlockSpec((BLK, D), lambda i: (N // BLK + i, 0)),
        pl.BlockSpec((D, H), lambda i: (0, 0)),
        pl.BlockSpec((1, H), lambda i: (0, 0)),
        pl.BlockSpec((H, H), lambda i: (0, 0)),
        pl.BlockSpec((1, H), lambda i: (0, 0)),
        pl.BlockSpec((H, D), lambda i: (0, 0)),
        pl.BlockSpec((1, D), lambda i: (0, 0)),
    ],
    out_specs=pl.BlockSpec((BLK, D), lambda i: (i, 0)),
    out_shape=jax.ShapeDtypeStruct((N, D), jnp.float32),
)


def kernel(edge_index, embed, eps0, W1_0, b1_0, W2_0, b2_0, W3_0, b3_0,
           eps1, W1_1, b1_1, W2_1, b2_1, W3_1, b3_1,
           eps2, W1_2, b1_2, W2_2, b2_2, W3_2, b3_2):
    zeros = jnp.zeros((ZR, D), jnp.float32)
    s2, d2 = _untile(edge_index)
    src = jnp.reshape(s2, (E,))
    dst = jnp.reshape(d2, (E,))
    x = embed
    params = [(eps0, W1_0, b1_0, W2_0, b2_0, W3_0, b3_0),
              (eps1, W1_1, b1_1, W2_1, b2_1, W3_1, b3_1),
              (eps2, W1_2, b1_2, W2_2, b2_2, W3_2, b3_2)]
    for eps, W1, b1, W2, b2, W3, b3 in params:
        parts = _sc_agg(src, dst, x, zeros)
        x = _mlp(jnp.reshape(eps, (1,)), x, parts, parts,
                 W1, jnp.reshape(b1, (1, H)),
                 W2, jnp.reshape(b2, (1, H)),
                 W3, jnp.reshape(b3, (1, D)))
    return x
